# baseline (speedup 1.0000x reference)
import functools

import jax
import jax.numpy as jnp
from jax import lax
from jax.experimental import pallas as pl
from jax.experimental.pallas import tpu as pltpu
from jax.experimental.pallas import tpu_sc as plsc

S = 8192
D = 768
K = 1024
P = 8192
NC = 2
NS = 16
L = 16
NB = D // 128
NQ = 4
QR = S // NQ
TR = QR // NS
EV = P // NS
BE = 32
CAP = EV + BE + L
KP = K + 8


def _proj_table_kernel(beta_ref, mark_ref, w_ref, b_ref, out_ref):
    acc = lax.dot_general(
        mark_ref[...], w_ref[...],
        dimension_numbers=(((1,), (1,)), ((), ())),
        preferred_element_type=jnp.float32,
    )
    beta = beta_ref[0, 0]
    res = beta * (acc + b_ref[...])
    for cc in range(NB):
        out_ref[cc, 0:K, :] = res[:, cc * 128:(cc + 1) * 128]
        out_ref[cc, K:KP, :] = jnp.zeros((KP - K, 128), jnp.float32)


def _sc_inject_kernel(text_hbm, proj_hbm, eid_hbm, pos_hbm, out_hbm,
                      eid_v, pos_v, gl, sl, gidx6, bidx6, rows6, acc6, sem):
    c = lax.axis_index("c")
    s = lax.axis_index("s")

    pltpu.sync_copy(eid_hbm.at[pl.ds(s * EV, EV)], eid_v)
    pltpu.sync_copy(pos_hbm.at[pl.ds(s * EV, EV)], pos_v)

    zero_v = jnp.zeros((L,), jnp.int32)
    one_v = jnp.ones((L,), jnp.int32)
    pad_v = jnp.full((L,), K, jnp.int32)
    trash_v = jnp.full((L,), CAP - 1, jnp.int32)

    for q in range(NQ // NC):
        base = (c * (NQ // NC) + q) * QR

        def init_or_writeout(acc, write):
            for cc in range(NB):
                a = acc.at[pl.ds(cc * QR + s * TR, TR)]
                h_src = text_hbm if not write else out_hbm
                h = h_src.at[pl.ds(base + s * TR, TR),
                             pl.ds(cc * 128, 128)]
                if write:
                    pltpu.sync_copy(a, h)
                else:
                    pltpu.sync_copy(h, a)

        init_or_writeout(acc6, write=False)

        lo_v = jnp.full((L,), base, jnp.int32)
        hi_v = jnp.full((L,), base + QR, jnp.int32)

        def scan_body(i, off):
            e = eid_v[pl.ds(i * L, L)]
            p = pos_v[pl.ds(i * L, L)]
            m = (e > zero_v) & (p >= lo_v) & (p < hi_v)
            mi = jnp.where(m, one_v, zero_v)
            off_v = jnp.full((L,), off, jnp.int32)
            dst = jnp.where(m, off_v + plsc.cumsum(mi) - mi, trash_v)
            plsc.store_scatter(gl, [dst], e - one_v)
            plsc.store_scatter(sl, [dst], p - lo_v)
            return off + jnp.sum(mi)

        cnt = lax.fori_loop(0, EV // L, scan_body, jnp.int32(0))

        for j in range(BE // L):
            gl[pl.ds(cnt + j * L, L)] = pad_v
            sl[pl.ds(cnt + j * L, L)] = zero_v
        nbat = (cnt + BE - 1) // BE

        plsc.subcore_barrier()

        def batch_body(b, carry):
            for j in range(BE // L):
                g = gl[pl.ds(b * BE + j * L, L)]
                t = sl[pl.ds(b * BE + j * L, L)]
                for cc in range(NB):
                    gidx6[cc, pl.ds(j * L, L)] = g + jnp.full(
                        (L,), cc * KP, jnp.int32)
                    bidx6[cc, pl.ds(j * L, L)] = t + jnp.full(
                        (L,), cc * QR, jnp.int32)
            copies = [
                pltpu.async_copy(proj_hbm.at[gidx6.at[cc]],
                                 rows6.at[cc], sem)
                for cc in range(NB)
            ]
            for cc in range(NB):
                copies[cc].wait()
                pltpu.sync_copy(rows6.at[cc], acc6.at[bidx6.at[cc]],
                                add=True)
            return carry

        lax.fori_loop(0, nbat, batch_body, jnp.int32(0))

        plsc.subcore_barrier()

        init_or_writeout(acc6, write=True)


def kernel(text_embeddings, mark_embeddings, entity_ids, positions, W, b, beta):
    proj = pl.pallas_call(
        _proj_table_kernel,
        out_shape=jax.ShapeDtypeStruct((NB, KP, 128), jnp.float32),
        in_specs=[
            pl.BlockSpec(memory_space=pltpu.SMEM),
            pl.BlockSpec(memory_space=pltpu.VMEM),
            pl.BlockSpec(memory_space=pltpu.VMEM),
            pl.BlockSpec(memory_space=pltpu.VMEM),
        ],
        out_specs=pl.BlockSpec(memory_space=pltpu.VMEM),
    )(jnp.reshape(beta, (1, 1)), mark_embeddings, W, jnp.reshape(b, (1, D)))
    proj = jnp.reshape(proj, (NB * KP, 128))

    mesh = plsc.VectorSubcoreMesh(core_axis_name="c", subcore_axis_name="s",
                                  num_cores=NC, num_subcores=NS)
    inject = functools.partial(
        pl.kernel,
        out_type=jax.ShapeDtypeStruct((S, D), jnp.float32),
        mesh=mesh,
        scratch_types=[
            pltpu.VMEM((EV,), jnp.int32),
            pltpu.VMEM((EV,), jnp.int32),
            pltpu.VMEM((CAP,), jnp.int32),
            pltpu.VMEM((CAP,), jnp.int32),
            pltpu.VMEM((NB, BE), jnp.int32),
            pltpu.VMEM((NB, BE), jnp.int32),
            pltpu.VMEM((NB, BE, 128), jnp.float32),
            pltpu.VMEM_SHARED((NB * QR, 128), jnp.float32),
            pltpu.SemaphoreType.DMA,
        ],
        compiler_params=pltpu.CompilerParams(needs_layout_passes=False),
    )(_sc_inject_kernel)

    return inject(text_embeddings, proj,
                  entity_ids.astype(jnp.int32), positions.astype(jnp.int32))


# speedup vs baseline: 1.0078x; 1.0078x over previous
import functools

import jax
import jax.numpy as jnp
from jax import lax
from jax.experimental import pallas as pl
from jax.experimental.pallas import tpu as pltpu
from jax.experimental.pallas import tpu_sc as plsc

S = 8192
D = 768
K = 1024
P = 8192
NC = 2
NS = 16
L = 16
NB = D // 128
NQ = 4
QR = S // NQ
TR = QR // NS
EV = P // NS
BE = 32
CAP = EV + BE + L
KP = K + 8


def _proj_table_kernel(beta_ref, mark_ref, w_ref, b_ref, out_ref):
    acc = lax.dot_general(
        mark_ref[...], w_ref[...],
        dimension_numbers=(((1,), (1,)), ((), ())),
        preferred_element_type=jnp.float32,
    )
    beta = beta_ref[0, 0]
    res = beta * (acc + b_ref[...])
    for cc in range(NB):
        out_ref[cc, 0:K, :] = res[:, cc * 128:(cc + 1) * 128]
        out_ref[cc, K:KP, :] = jnp.zeros((KP - K, 128), jnp.float32)


def _sc_inject_kernel(text_hbm, proj_hbm, eid_hbm, pos_hbm, out_hbm,
                      eid_v, pos_v, gl, sl, gidx6, bidx6, rows6, acc6, sem):
    c = lax.axis_index("c")
    s = lax.axis_index("s")

    pltpu.sync_copy(eid_hbm.at[pl.ds(s * EV, EV)], eid_v)
    pltpu.sync_copy(pos_hbm.at[pl.ds(s * EV, EV)], pos_v)

    zero_v = jnp.zeros((L,), jnp.int32)
    one_v = jnp.ones((L,), jnp.int32)
    pad_v = jnp.full((L,), K, jnp.int32)
    trash_v = jnp.full((L,), CAP - 1, jnp.int32)

    for q in range(NQ // NC):
        base = (c * (NQ // NC) + q) * QR

        def init_or_writeout(acc, write):
            for cc in range(NB):
                a = acc.at[pl.ds(cc * QR + s * TR, TR)]
                h_src = text_hbm if not write else out_hbm
                h = h_src.at[pl.ds(base + s * TR, TR),
                             pl.ds(cc * 128, 128)]
                if write:
                    pltpu.sync_copy(a, h)
                else:
                    pltpu.sync_copy(h, a)

        icopies = [
            pltpu.async_copy(
                text_hbm.at[pl.ds(base + s * TR, TR),
                            pl.ds(cc * 128, 128)],
                acc6.at[pl.ds(cc * QR + s * TR, TR)], sem)
            for cc in range(NB)
        ]

        lo_v = jnp.full((L,), base, jnp.int32)
        hi_v = jnp.full((L,), base + QR, jnp.int32)

        def scan_body(i, off):
            e = eid_v[pl.ds(i * L, L)]
            p = pos_v[pl.ds(i * L, L)]
            m = (e > zero_v) & (p >= lo_v) & (p < hi_v)
            mi = jnp.where(m, one_v, zero_v)
            off_v = jnp.full((L,), off, jnp.int32)
            dst = jnp.where(m, off_v + plsc.cumsum(mi) - mi, trash_v)
            plsc.store_scatter(gl, [dst], e - one_v)
            plsc.store_scatter(sl, [dst], p - lo_v)
            return off + jnp.sum(mi)

        cnt = lax.fori_loop(0, EV // L, scan_body, jnp.int32(0))

        for j in range(BE // L):
            gl[pl.ds(cnt + j * L, L)] = pad_v
            sl[pl.ds(cnt + j * L, L)] = zero_v
        nbat = (cnt + BE - 1) // BE

        for d in icopies:
            d.wait()
        plsc.subcore_barrier()

        def batch_body(b, carry):
            for j in range(BE // L):
                g = gl[pl.ds(b * BE + j * L, L)]
                t = sl[pl.ds(b * BE + j * L, L)]
                for cc in range(NB):
                    gidx6[cc, pl.ds(j * L, L)] = g + jnp.full(
                        (L,), cc * KP, jnp.int32)
                    bidx6[cc, pl.ds(j * L, L)] = t + jnp.full(
                        (L,), cc * QR, jnp.int32)
            copies = [
                pltpu.async_copy(proj_hbm.at[gidx6.at[cc]],
                                 rows6.at[cc], sem)
                for cc in range(NB)
            ]
            for cc in range(NB):
                copies[cc].wait()
                pltpu.sync_copy(rows6.at[cc], acc6.at[bidx6.at[cc]],
                                add=True)
            return carry

        lax.fori_loop(0, nbat, batch_body, jnp.int32(0))

        plsc.subcore_barrier()

        init_or_writeout(acc6, write=True)


def kernel(text_embeddings, mark_embeddings, entity_ids, positions, W, b, beta):
    proj = pl.pallas_call(
        _proj_table_kernel,
        out_shape=jax.ShapeDtypeStruct((NB, KP, 128), jnp.float32),
        in_specs=[
            pl.BlockSpec(memory_space=pltpu.SMEM),
            pl.BlockSpec(memory_space=pltpu.VMEM),
            pl.BlockSpec(memory_space=pltpu.VMEM),
            pl.BlockSpec(memory_space=pltpu.VMEM),
        ],
        out_specs=pl.BlockSpec(memory_space=pltpu.VMEM),
    )(jnp.reshape(beta, (1, 1)), mark_embeddings, W, jnp.reshape(b, (1, D)))
    proj = jnp.reshape(proj, (NB * KP, 128))

    mesh = plsc.VectorSubcoreMesh(core_axis_name="c", subcore_axis_name="s",
                                  num_cores=NC, num_subcores=NS)
    inject = functools.partial(
        pl.kernel,
        out_type=jax.ShapeDtypeStruct((S, D), jnp.float32),
        mesh=mesh,
        scratch_types=[
            pltpu.VMEM((EV,), jnp.int32),
            pltpu.VMEM((EV,), jnp.int32),
            pltpu.VMEM((CAP,), jnp.int32),
            pltpu.VMEM((CAP,), jnp.int32),
            pltpu.VMEM((NB, BE), jnp.int32),
            pltpu.VMEM((NB, BE), jnp.int32),
            pltpu.VMEM((NB, BE, 128), jnp.float32),
            pltpu.VMEM_SHARED((NB * QR, 128), jnp.float32),
            pltpu.SemaphoreType.DMA,
        ],
        compiler_params=pltpu.CompilerParams(needs_layout_passes=False),
    )(_sc_inject_kernel)

    return inject(text_embeddings, proj,
                  entity_ids.astype(jnp.int32), positions.astype(jnp.int32))
